# baseline (speedup 1.0000x reference)
import functools

import jax
import jax.numpy as jnp
from jax import lax
from jax.experimental import pallas as pl
from jax.experimental.pallas import tpu as pltpu
from jax.experimental.pallas import tpu_sc as plsc

NC = 2
NS = 16
NW = NC * NS
C = 128
D = 100
DP = 128
NBUF = 4

PAD_BR = 1024
SLICE_BR = 2048


@functools.lru_cache(maxsize=None)
def _build_gather(n_rows, n_chunks):
    mesh = plsc.VectorSubcoreMesh(core_axis_name="c", subcore_axis_name="s")
    per_w = n_chunks * C

    @functools.partial(
        pl.kernel,
        mesh=mesh,
        out_type=jax.ShapeDtypeStruct((n_rows, DP), jnp.float32),
        compiler_params=pltpu.CompilerParams(use_tc_tiling_on_sc=False),
        scratch_types=(
            [pltpu.VMEM((n_chunks, C), jnp.int32)]
            + [pltpu.VMEM((C, DP), jnp.float32) for _ in range(NBUF)]
            + [pltpu.SemaphoreType.DMA for _ in range(2 * NBUF)]
        ),
    )
    def k(table_hbm, idx_hbm, out_hbm, idx_v, *rest):
        rows = list(rest[:NBUF])
        gsem = list(rest[NBUF:2 * NBUF])
        osem = list(rest[2 * NBUF:])
        wid = lax.axis_index("s") * NC + lax.axis_index("c")
        base = wid * per_w

        pltpu.sync_copy(idx_hbm.at[wid], idx_v)

        def g_copy(j, b):
            return pltpu.make_async_copy(
                table_hbm.at[idx_v.at[j]], rows[b], gsem[b])

        def s_copy(j, b):
            return pltpu.make_async_copy(
                rows[b], out_hbm.at[pl.ds(base + j * C, C)], osem[b])

        g_copy(0, 0).start()
        g_copy(1, 1).start()

        for j in range(NBUF):
            if j >= 2:
                s_copy(j - 2, (j - 2) % NBUF).wait()
            g_copy(j + 2, (j + 2) % NBUF).start()
            g_copy(j, j).wait()
            s_copy(j, j).start()

        def group(g, carry):
            for b in range(NBUF):
                j = g * NBUF + b
                s_copy(j - 2, (b - 2) % NBUF).wait()
                g_copy(j + 2, (b + 2) % NBUF).start()
                g_copy(j, b).wait()
                s_copy(j, b).start()
            return carry

        lax.fori_loop(1, n_chunks // NBUF - 1, group, None)

        for b in range(NBUF):
            j = n_chunks - NBUF + b
            s_copy(j - 2, (j - 2) % NBUF).wait()
            if j + 2 < n_chunks:
                g_copy(j + 2, (j + 2) % NBUF).start()
            g_copy(j, b).wait()
            s_copy(j, b).start()

        for j in (n_chunks - 2, n_chunks - 1):
            s_copy(j, j % NBUF).wait()

    return k


def _pad_body(x_ref, o_ref):
    o_ref[...] = jnp.concatenate(
        [x_ref[...], jnp.zeros((x_ref.shape[0], DP - D), jnp.float32)], axis=1)


@functools.lru_cache(maxsize=None)
def _build_pad(n_rows):
    grid = (pl.cdiv(n_rows, PAD_BR),)
    return pl.pallas_call(
        _pad_body,
        grid=grid,
        in_specs=[pl.BlockSpec((PAD_BR, D), lambda i: (i, 0))],
        out_specs=pl.BlockSpec((PAD_BR, DP), lambda i: (i, 0)),
        out_shape=jax.ShapeDtypeStruct((n_rows, DP), jnp.float32),
    )


def kernel(inputs, embeddings):
    n_rows = inputs.size
    table = jnp.pad(embeddings, ((0, 0), (0, DP - D)))
    idx = inputs.reshape(NW, -1, C).astype(jnp.int32)
    out = _build_gather(n_rows, idx.shape[1])(table, idx)
    return out[:, :D].reshape(*inputs.shape, D)


# speedup vs baseline: 1.1130x; 1.1130x over previous
import functools

import jax
import jax.numpy as jnp
from jax import lax
from jax.experimental import pallas as pl
from jax.experimental.pallas import tpu as pltpu
from jax.experimental.pallas import tpu_sc as plsc

NC = 2
NS = 16
NW = NC * NS
C = 128
D = 100
DP = 128
NBUF = 4

PAD_BR = 1024
SLICE_BR = 2048


@functools.lru_cache(maxsize=None)
def _build_gather(n_rows, n_chunks):
    mesh = plsc.VectorSubcoreMesh(core_axis_name="c", subcore_axis_name="s")
    per_w = n_chunks * C

    @functools.partial(
        pl.kernel,
        mesh=mesh,
        out_type=jax.ShapeDtypeStruct((n_rows, DP), jnp.float32),
        compiler_params=pltpu.CompilerParams(use_tc_tiling_on_sc=False),
        scratch_types=(
            [pltpu.VMEM((n_chunks, C), jnp.int32)]
            + [pltpu.VMEM((C, DP), jnp.float32) for _ in range(NBUF)]
            + [pltpu.SemaphoreType.DMA for _ in range(2 * NBUF)]
        ),
    )
    def k(table_hbm, idx_hbm, out_hbm, idx_v, *rest):
        rows = list(rest[:NBUF])
        gsem = list(rest[NBUF:2 * NBUF])
        osem = list(rest[2 * NBUF:])
        wid = lax.axis_index("s") * NC + lax.axis_index("c")
        base = wid * per_w

        pltpu.sync_copy(idx_hbm.at[wid], idx_v)

        def g_copy(j, b):
            return pltpu.make_async_copy(
                table_hbm.at[idx_v.at[j]], rows[b], gsem[b])

        def s_copy(j, b):
            return pltpu.make_async_copy(
                rows[b], out_hbm.at[pl.ds(base + j * C, C)], osem[b])

        g_copy(0, 0).start()
        g_copy(1, 1).start()

        for j in range(NBUF):
            if j >= 2:
                s_copy(j - 2, (j - 2) % NBUF).wait()
            g_copy(j + 2, (j + 2) % NBUF).start()
            g_copy(j, j).wait()
            s_copy(j, j).start()

        def group(g, carry):
            for b in range(NBUF):
                j = g * NBUF + b
                s_copy(j - 2, (b - 2) % NBUF).wait()
                g_copy(j + 2, (b + 2) % NBUF).start()
                g_copy(j, b).wait()
                s_copy(j, b).start()
            return carry

        lax.fori_loop(1, n_chunks // NBUF - 1, group, None)

        for b in range(NBUF):
            j = n_chunks - NBUF + b
            s_copy(j - 2, (j - 2) % NBUF).wait()
            if j + 2 < n_chunks:
                g_copy(j + 2, (j + 2) % NBUF).start()
            g_copy(j, b).wait()
            s_copy(j, b).start()

        for j in (n_chunks - 2, n_chunks - 1):
            s_copy(j, j % NBUF).wait()

    return k


def _pad_body(x_ref, o_ref):
    o_ref[...] = jnp.concatenate(
        [x_ref[...], jnp.zeros((x_ref.shape[0], DP - D), jnp.float32)], axis=1)


@functools.lru_cache(maxsize=None)
def _build_pad(n_rows):
    grid = (pl.cdiv(n_rows, PAD_BR),)
    return pl.pallas_call(
        _pad_body,
        grid=grid,
        in_specs=[pl.BlockSpec((PAD_BR, D), lambda i: (i, 0))],
        out_specs=pl.BlockSpec((PAD_BR, DP), lambda i: (i, 0)),
        out_shape=jax.ShapeDtypeStruct((n_rows, DP), jnp.float32),
    )


def kernel(inputs, embeddings):
    n_rows = inputs.size
    table = _build_pad(embeddings.shape[0])(embeddings)
    idx = inputs.reshape(NW, -1, C).astype(jnp.int32)
    out = _build_gather(n_rows, idx.shape[1])(table, idx)
    return out[:, :D].reshape(*inputs.shape, D)
